# Initial kernel scaffold; baseline (speedup 1.0000x reference)
import jax
import jax.numpy as jnp
from jax.experimental import pallas as pl


def kernel(x, edge_index, W1, b1, W2, b2, W3, b3, W4, b4):
    raise NotImplementedError("write your pallas kernel here")


# speedup vs baseline: 5.1244x; 5.1244x over previous
import functools

import jax
import jax.numpy as jnp
from jax import lax
from jax.experimental import pallas as pl
from jax.experimental.pallas import tpu as pltpu
from jax.experimental.pallas import tpu_sc as plsc

N = 10000
E = 320000
NC, NS = 2, 16
NW = NC * NS
EPW = E // NW
CH = 128
NFULL = EPW // CH
TAIL = EPW - NFULL * CH
RPS = 624
REXTRA = N - NS * RPS


def _make_agg(D):
    mesh = plsc.VectorSubcoreMesh(
        core_axis_name="c", subcore_axis_name="s",
        num_cores=NC, num_subcores=NS)

    @functools.partial(
        pl.kernel,
        out_type=jax.ShapeDtypeStruct((NC, N, D), jnp.float32),
        mesh=mesh,
        scratch_types=[
            pltpu.VMEM((CH,), jnp.int32),
            pltpu.VMEM((CH,), jnp.int32),
            pltpu.VMEM((CH, D), jnp.float32),
            pltpu.VMEM((TAIL,), jnp.int32),
            pltpu.VMEM((TAIL,), jnp.int32),
            pltpu.VMEM((TAIL, D), jnp.float32),
            pltpu.VMEM((RPS, D), jnp.float32),
            pltpu.VMEM((REXTRA, D), jnp.float32),
            pltpu.VMEM_SHARED((N, D), jnp.float32),
            pltpu.SemaphoreType.DMA,
        ],
        compiler_params=pltpu.CompilerParams(use_tc_tiling_on_sc=False),
    )
    def agg(x_hbm, src_hbm, dst_hbm, z_hbm, out_hbm,
            src_v, dst_v, rows_v, srct_v, dstt_v, rowst_v, zbuf, xbuf,
            acc, sem):
        c = lax.axis_index("c")
        s = lax.axis_index("s")
        wid = c * NS + s
        r0 = s * RPS
        pltpu.sync_copy(z_hbm.at[pl.ds(0, RPS)], zbuf)
        pltpu.sync_copy(zbuf, acc.at[pl.ds(r0, RPS)])

        @pl.when(s == NS - 1)
        def _():
            pltpu.sync_copy(z_hbm.at[pl.ds(NS * RPS, REXTRA)], xbuf)
            pltpu.sync_copy(xbuf, acc.at[pl.ds(NS * RPS, REXTRA)])

        plsc.subcore_barrier()

        base = wid * EPW

        def body(i, carry):
            off = base + i * CH
            pltpu.sync_copy(src_hbm.at[pl.ds(off, CH)], src_v)
            pltpu.sync_copy(dst_hbm.at[pl.ds(off, CH)], dst_v)
            pltpu.async_copy(x_hbm.at[src_v], rows_v, sem).wait()
            pltpu.sync_copy(rows_v, acc.at[dst_v], add=True)
            return carry

        lax.fori_loop(0, NFULL, body, 0)

        offt = base + NFULL * CH
        pltpu.sync_copy(src_hbm.at[pl.ds(offt, TAIL)], srct_v)
        pltpu.sync_copy(dst_hbm.at[pl.ds(offt, TAIL)], dstt_v)
        pltpu.async_copy(x_hbm.at[srct_v], rowst_v, sem).wait()
        pltpu.sync_copy(rowst_v, acc.at[dstt_v], add=True)

        plsc.subcore_barrier()
        pltpu.sync_copy(acc.at[pl.ds(r0, RPS)], zbuf)
        pltpu.sync_copy(zbuf, out_hbm.at[c, pl.ds(r0, RPS)])

        @pl.when(s == NS - 1)
        def _():
            pltpu.sync_copy(acc.at[pl.ds(NS * RPS, REXTRA)], xbuf)
            pltpu.sync_copy(xbuf, out_hbm.at[c, pl.ds(NS * RPS, REXTRA)])

    return agg


_agg64 = _make_agg(64)
_agg16 = _make_agg(16)

_BLK = 1000
_GRID = N // _BLK


def _tc1_body(pa_ref, pb_ref, w_ref, b_ref, oa_ref, ob_ref):
    pa = pa_ref[0] + pa_ref[1]
    pb = pb_ref[0] + pb_ref[1]
    h = (jnp.dot(pa, w_ref[:64], preferred_element_type=jnp.float32)
         + jnp.dot(pb, w_ref[64:], preferred_element_type=jnp.float32)
         + b_ref[...])
    h = jnp.maximum(h, 0.0)
    oa_ref[...] = h[:, :64]
    ob_ref[...] = h[:, 64:]


def _tc2_body(pa_ref, pb_ref, w2_ref, b2_ref, w3_ref, o_ref):
    pa = pa_ref[0] + pa_ref[1]
    pb = pb_ref[0] + pb_ref[1]
    h = jnp.maximum(
        jnp.dot(pa, w2_ref[:64], preferred_element_type=jnp.float32)
        + jnp.dot(pb, w2_ref[64:], preferred_element_type=jnp.float32)
        + b2_ref[...], 0.0)
    o_ref[...] = jnp.dot(h, w3_ref[...], preferred_element_type=jnp.float32)


def _tc3_body(p_ref, b3_ref, w4_ref, o_ref):
    t = jnp.maximum(p_ref[0] + p_ref[1] + b3_ref[...], 0.0)
    o_ref[...] = jnp.dot(t, w4_ref[...], preferred_element_type=jnp.float32)


def _tc4_body(p_ref, b4_ref, o_ref):
    o_ref[...] = (p_ref[0] + p_ref[1])[:, :2] + b4_ref[...]


def _full(shape):
    return pl.BlockSpec(shape, lambda i: tuple(0 for _ in shape))


def kernel(x, edge_index, W1, b1, W2, b2, W3, b3, W4, b4):
    src = edge_index[0]
    dst = edge_index[1]
    z64 = jnp.zeros((N, 64), jnp.float32)
    z16 = jnp.zeros((N, 16), jnp.float32)
    W4p = jnp.pad(W4, ((0, 0), (0, 16 - W4.shape[1])))
    b1r, b2r, b3r, b4r = (b.reshape(1, -1) for b in (b1, b2, b3, b4))

    Pa1 = _agg64(x[:, :64], src, dst, z64)
    Pb1 = _agg64(x[:, 64:], src, dst, z64)
    h1a, h1b = pl.pallas_call(
        _tc1_body,
        grid=(_GRID,),
        in_specs=[
            pl.BlockSpec((NC, _BLK, 64), lambda i: (0, i, 0)),
            pl.BlockSpec((NC, _BLK, 64), lambda i: (0, i, 0)),
            _full((128, 128)),
            _full((1, 128)),
        ],
        out_specs=[
            pl.BlockSpec((_BLK, 64), lambda i: (i, 0)),
            pl.BlockSpec((_BLK, 64), lambda i: (i, 0)),
        ],
        out_shape=[
            jax.ShapeDtypeStruct((N, 64), jnp.float32),
            jax.ShapeDtypeStruct((N, 64), jnp.float32),
        ],
    )(Pa1, Pb1, W1, b1r)

    Pa2 = _agg64(h1a, src, dst, z64)
    Pb2 = _agg64(h1b, src, dst, z64)
    t3 = pl.pallas_call(
        _tc2_body,
        grid=(_GRID,),
        in_specs=[
            pl.BlockSpec((NC, _BLK, 64), lambda i: (0, i, 0)),
            pl.BlockSpec((NC, _BLK, 64), lambda i: (0, i, 0)),
            _full((128, 128)),
            _full((1, 128)),
            _full((128, 64)),
        ],
        out_specs=pl.BlockSpec((_BLK, 64), lambda i: (i, 0)),
        out_shape=jax.ShapeDtypeStruct((N, 64), jnp.float32),
    )(Pa2, Pb2, W2, b2r, W3)

    P3 = _agg64(t3, src, dst, z64)
    t4 = pl.pallas_call(
        _tc3_body,
        grid=(_GRID,),
        in_specs=[
            pl.BlockSpec((NC, _BLK, 64), lambda i: (0, i, 0)),
            _full((1, 64)),
            _full((64, 16)),
        ],
        out_specs=pl.BlockSpec((_BLK, 16), lambda i: (i, 0)),
        out_shape=jax.ShapeDtypeStruct((N, 16), jnp.float32),
    )(P3, b3r, W4p)

    P4 = _agg16(t4, src, dst, z16)
    out = pl.pallas_call(
        _tc4_body,
        grid=(_GRID,),
        in_specs=[
            pl.BlockSpec((NC, _BLK, 16), lambda i: (0, i, 0)),
            _full((1, 2)),
        ],
        out_specs=pl.BlockSpec((_BLK, 2), lambda i: (i, 0)),
        out_shape=jax.ShapeDtypeStruct((N, 2), jnp.float32),
    )(P4, b4r)
    return out


# speedup vs baseline: 11.5291x; 2.2499x over previous
import functools

import jax
import jax.numpy as jnp
from jax import lax
from jax.experimental import pallas as pl
from jax.experimental.pallas import tpu as pltpu
from jax.experimental.pallas import tpu_sc as plsc

N = 10000
E = 320000
NC, NS = 2, 16
NW = NC * NS
CH = 128
NCH = E // CH
CPW = NCH // NW
EXTRA = NCH - NW * CPW
NBUF = 2
NITER = (CPW + NBUF - 1) // NBUF
RPS = 624
REXTRA = N - NS * RPS


def _make_agg(D):
    mesh = plsc.VectorSubcoreMesh(
        core_axis_name="c", subcore_axis_name="s",
        num_cores=NC, num_subcores=NS)

    @functools.partial(
        pl.kernel,
        out_type=jax.ShapeDtypeStruct((NC, N, D), jnp.float32),
        mesh=mesh,
        scratch_types=[
            pltpu.VMEM((CPW, CH), jnp.int32),
            pltpu.VMEM((CPW, CH), jnp.int32),
            pltpu.VMEM((1, CH), jnp.int32),
            pltpu.VMEM((1, CH), jnp.int32),
            [pltpu.VMEM((CH, D), jnp.float32) for _ in range(NBUF)],
            [pltpu.SemaphoreType.DMA for _ in range(NBUF)],
            pltpu.VMEM((RPS, D), jnp.float32),
            pltpu.VMEM((REXTRA, D), jnp.float32),
            pltpu.VMEM_SHARED((N, D), jnp.float32),
        ],
        compiler_params=pltpu.CompilerParams(use_tc_tiling_on_sc=False),
    )
    def agg(x_hbm, src_hbm, dst_hbm, z_hbm, out_hbm,
            srcs, dsts, exs, exd, bufs, gsems, zbuf, xbuf, acc):
        c = lax.axis_index("c")
        s = lax.axis_index("s")
        wid = c * NS + s
        pltpu.sync_copy(src_hbm.at[pl.ds(wid * CPW, CPW)], srcs)
        pltpu.sync_copy(dst_hbm.at[pl.ds(wid * CPW, CPW)], dsts)
        for k in range(NBUF):
            pltpu.async_copy(x_hbm.at[srcs.at[k]], bufs[k], gsems[k])
        r0 = s * RPS
        pltpu.sync_copy(z_hbm.at[pl.ds(0, RPS)], zbuf)
        pltpu.sync_copy(zbuf, acc.at[pl.ds(r0, RPS)])

        @pl.when(s == NS - 1)
        def _():
            pltpu.sync_copy(z_hbm.at[pl.ds(RPS, REXTRA)], xbuf)
            pltpu.sync_copy(xbuf, acc.at[pl.ds(NS * RPS, REXTRA)])

        plsc.subcore_barrier()

        def body(i, carry):
            for k in range(NBUF):
                cid = i * NBUF + k

                @pl.when(cid < CPW)
                def _():
                    pltpu.make_async_copy(
                        x_hbm.at[srcs.at[0]], bufs[k], gsems[k]).wait()
                    pltpu.sync_copy(bufs[k], acc.at[dsts.at[cid]], add=True)

                    @pl.when(cid + NBUF < CPW)
                    def _():
                        pltpu.async_copy(
                            x_hbm.at[srcs.at[cid + NBUF]], bufs[k], gsems[k])
            return carry

        lax.fori_loop(0, NITER, body, 0)

        @pl.when(wid < EXTRA)
        def _():
            pltpu.sync_copy(src_hbm.at[pl.ds(NW * CPW + wid, 1)], exs)
            pltpu.sync_copy(dst_hbm.at[pl.ds(NW * CPW + wid, 1)], exd)
            pltpu.async_copy(x_hbm.at[exs.at[0]], bufs[0], gsems[0]).wait()
            pltpu.sync_copy(bufs[0], acc.at[exd.at[0]], add=True)

        plsc.subcore_barrier()
        pltpu.sync_copy(acc.at[pl.ds(r0, RPS)], zbuf)
        pltpu.sync_copy(zbuf, out_hbm.at[c, pl.ds(r0, RPS)])

        @pl.when(s == NS - 1)
        def _():
            pltpu.sync_copy(acc.at[pl.ds(NS * RPS, REXTRA)], xbuf)
            pltpu.sync_copy(xbuf, out_hbm.at[c, pl.ds(NS * RPS, REXTRA)])

    return agg


_agg64 = _make_agg(64)
_agg16 = _make_agg(16)

_BLK = 1000
_GRID = N // _BLK


def _tc1_body(pa_ref, pb_ref, w_ref, b_ref, oa_ref, ob_ref):
    pa = pa_ref[0] + pa_ref[1]
    pb = pb_ref[0] + pb_ref[1]
    h = (jnp.dot(pa, w_ref[:64], preferred_element_type=jnp.float32)
         + jnp.dot(pb, w_ref[64:], preferred_element_type=jnp.float32)
         + b_ref[...])
    h = jnp.maximum(h, 0.0)
    oa_ref[...] = h[:, :64]
    ob_ref[...] = h[:, 64:]


def _tc2_body(pa_ref, pb_ref, w2_ref, b2_ref, w3_ref, o_ref):
    pa = pa_ref[0] + pa_ref[1]
    pb = pb_ref[0] + pb_ref[1]
    h = jnp.maximum(
        jnp.dot(pa, w2_ref[:64], preferred_element_type=jnp.float32)
        + jnp.dot(pb, w2_ref[64:], preferred_element_type=jnp.float32)
        + b2_ref[...], 0.0)
    o_ref[...] = jnp.dot(h, w3_ref[...], preferred_element_type=jnp.float32)


def _tc3_body(p_ref, b3_ref, w4_ref, o_ref):
    t = jnp.maximum(p_ref[0] + p_ref[1] + b3_ref[...], 0.0)
    o_ref[...] = jnp.dot(t, w4_ref[...], preferred_element_type=jnp.float32)


def _tc4_body(p_ref, b4_ref, o_ref):
    o_ref[...] = (p_ref[0] + p_ref[1])[:, :2] + b4_ref[...]


def _full(shape):
    return pl.BlockSpec(shape, lambda i: tuple(0 for _ in shape))


def kernel(x, edge_index, W1, b1, W2, b2, W3, b3, W4, b4):
    src = edge_index[0].reshape(NCH, CH)
    dst = edge_index[1].reshape(NCH, CH)
    z64 = jnp.zeros((RPS + REXTRA, 64), jnp.float32)
    z16 = jnp.zeros((RPS + REXTRA, 16), jnp.float32)
    W4p = jnp.pad(W4, ((0, 0), (0, 16 - W4.shape[1])))
    b1r, b2r, b3r, b4r = (b.reshape(1, -1) for b in (b1, b2, b3, b4))

    Pa1 = _agg64(x[:, :64], src, dst, z64)
    Pb1 = _agg64(x[:, 64:], src, dst, z64)
    h1a, h1b = pl.pallas_call(
        _tc1_body,
        grid=(_GRID,),
        in_specs=[
            pl.BlockSpec((NC, _BLK, 64), lambda i: (0, i, 0)),
            pl.BlockSpec((NC, _BLK, 64), lambda i: (0, i, 0)),
            _full((128, 128)),
            _full((1, 128)),
        ],
        out_specs=[
            pl.BlockSpec((_BLK, 64), lambda i: (i, 0)),
            pl.BlockSpec((_BLK, 64), lambda i: (i, 0)),
        ],
        out_shape=[
            jax.ShapeDtypeStruct((N, 64), jnp.float32),
            jax.ShapeDtypeStruct((N, 64), jnp.float32),
        ],
    )(Pa1, Pb1, W1, b1r)

    Pa2 = _agg64(h1a, src, dst, z64)
    Pb2 = _agg64(h1b, src, dst, z64)
    t3 = pl.pallas_call(
        _tc2_body,
        grid=(_GRID,),
        in_specs=[
            pl.BlockSpec((NC, _BLK, 64), lambda i: (0, i, 0)),
            pl.BlockSpec((NC, _BLK, 64), lambda i: (0, i, 0)),
            _full((128, 128)),
            _full((1, 128)),
            _full((128, 64)),
        ],
        out_specs=pl.BlockSpec((_BLK, 64), lambda i: (i, 0)),
        out_shape=jax.ShapeDtypeStruct((N, 64), jnp.float32),
    )(Pa2, Pb2, W2, b2r, W3)

    P3 = _agg64(t3, src, dst, z64)
    t4 = pl.pallas_call(
        _tc3_body,
        grid=(_GRID,),
        in_specs=[
            pl.BlockSpec((NC, _BLK, 64), lambda i: (0, i, 0)),
            _full((1, 64)),
            _full((64, 16)),
        ],
        out_specs=pl.BlockSpec((_BLK, 16), lambda i: (i, 0)),
        out_shape=jax.ShapeDtypeStruct((N, 16), jnp.float32),
    )(P3, b3r, W4p)

    P4 = _agg16(t4, src, dst, z16)
    out = pl.pallas_call(
        _tc4_body,
        grid=(_GRID,),
        in_specs=[
            pl.BlockSpec((NC, _BLK, 16), lambda i: (0, i, 0)),
            _full((1, 2)),
        ],
        out_specs=pl.BlockSpec((_BLK, 2), lambda i: (i, 0)),
        out_shape=jax.ShapeDtypeStruct((N, 2), jnp.float32),
    )(P4, b4r)
    return out


# speedup vs baseline: 14.1428x; 1.2267x over previous
import functools

import jax
import jax.numpy as jnp
from jax import lax
from jax.experimental import pallas as pl
from jax.experimental.pallas import tpu as pltpu
from jax.experimental.pallas import tpu_sc as plsc

N = 10000
E = 320000
NC, NS = 2, 16
NW = NC * NS
CH = 128
NCH = E // CH
CPW = NCH // NW
EXTRA = NCH - NW * CPW
NBUF = 2
NHALF = 2
CPH = CPW // NHALF
NITER = (CPH + NBUF - 1) // NBUF
RPS = 624
REXTRA = N - NS * RPS


def _make_agg(D):
    mesh = plsc.VectorSubcoreMesh(
        core_axis_name="c", subcore_axis_name="s",
        num_cores=NC, num_subcores=NS)

    @functools.partial(
        pl.kernel,
        out_type=jax.ShapeDtypeStruct((NC, N, D), jnp.float32),
        mesh=mesh,
        scratch_types=[
            pltpu.VMEM((CPH, CH), jnp.int32),
            pltpu.VMEM((CPH, CH), jnp.int32),
            [pltpu.VMEM((CH, D), jnp.float32) for _ in range(NBUF)],
            [pltpu.SemaphoreType.DMA for _ in range(NBUF)],
            pltpu.VMEM_SHARED((N, D), jnp.float32),
        ],
        compiler_params=pltpu.CompilerParams(use_tc_tiling_on_sc=False),
    )
    def agg(x_hbm, src_hbm, dst_hbm, z_hbm, out_hbm,
            srcs, dsts, bufs, gsems, acc):
        c = lax.axis_index("c")
        s = lax.axis_index("s")
        wid = c * NS + s
        r0 = s * RPS
        pltpu.sync_copy(z_hbm.at[pl.ds(0, RPS)], acc.at[pl.ds(r0, RPS)])

        @pl.when(s == NS - 1)
        def _():
            pltpu.sync_copy(z_hbm.at[pl.ds(RPS, REXTRA)],
                            acc.at[pl.ds(NS * RPS, REXTRA)])

        def run_bank(first):
            def body(i, carry):
                for k in range(NBUF):
                    cid = i * NBUF + k

                    @pl.when(cid < CPH)
                    def _():
                        pltpu.make_async_copy(
                            x_hbm.at[srcs.at[0]], bufs[k], gsems[k]).wait()
                        pltpu.sync_copy(bufs[k], acc.at[dsts.at[cid]],
                                        add=True)

                        @pl.when(cid + NBUF < CPH)
                        def _():
                            pltpu.async_copy(
                                x_hbm.at[srcs.at[cid + NBUF]],
                                bufs[k], gsems[k])
                return carry
            return body

        for half in range(NHALF):
            row0 = wid * CPW + half * CPH
            pltpu.sync_copy(src_hbm.at[pl.ds(row0, CPH)], srcs)
            pltpu.sync_copy(dst_hbm.at[pl.ds(row0, CPH)], dsts)
            for k in range(NBUF):
                pltpu.async_copy(x_hbm.at[srcs.at[k]], bufs[k], gsems[k])
            if half == 0:
                plsc.subcore_barrier()
            lax.fori_loop(0, NITER, run_bank(half), 0, unroll=False)

        @pl.when(wid < EXTRA)
        def _():
            pltpu.sync_copy(src_hbm.at[pl.ds(NW * CPW + wid, 1)],
                            srcs.at[pl.ds(0, 1)])
            pltpu.sync_copy(dst_hbm.at[pl.ds(NW * CPW + wid, 1)],
                            dsts.at[pl.ds(0, 1)])
            pltpu.async_copy(x_hbm.at[srcs.at[0]], bufs[0], gsems[0]).wait()
            pltpu.sync_copy(bufs[0], acc.at[dsts.at[0]], add=True)

        plsc.subcore_barrier()
        pltpu.sync_copy(acc.at[pl.ds(r0, RPS)], out_hbm.at[c, pl.ds(r0, RPS)])

        @pl.when(s == NS - 1)
        def _():
            pltpu.sync_copy(acc.at[pl.ds(NS * RPS, REXTRA)],
                            out_hbm.at[c, pl.ds(NS * RPS, REXTRA)])

    return agg


_agg128 = _make_agg(128)
_agg64 = _make_agg(64)
_agg16 = _make_agg(16)

_BLK = 1000
_GRID = N // _BLK


def _tc1_body(p_ref, w_ref, b_ref, o_ref):
    p = p_ref[0] + p_ref[1]
    h = jnp.dot(p, w_ref[...], preferred_element_type=jnp.float32) + b_ref[...]
    o_ref[...] = jnp.maximum(h, 0.0)


def _tc2_body(p_ref, w2_ref, b2_ref, w3_ref, o_ref):
    p = p_ref[0] + p_ref[1]
    h = jnp.maximum(
        jnp.dot(p, w2_ref[...], preferred_element_type=jnp.float32)
        + b2_ref[...], 0.0)
    o_ref[...] = jnp.dot(h, w3_ref[...], preferred_element_type=jnp.float32)


def _tc3_body(p_ref, b3_ref, w4_ref, o_ref):
    t = jnp.maximum(p_ref[0] + p_ref[1] + b3_ref[...], 0.0)
    o_ref[...] = jnp.dot(t, w4_ref[...], preferred_element_type=jnp.float32)


def _tc4_body(p_ref, b4_ref, o_ref):
    o_ref[...] = (p_ref[0] + p_ref[1])[:, :2] + b4_ref[...]


def _full(shape):
    return pl.BlockSpec(shape, lambda i: tuple(0 for _ in shape))


def kernel(x, edge_index, W1, b1, W2, b2, W3, b3, W4, b4):
    src = edge_index[0].reshape(NCH, CH)
    dst = edge_index[1].reshape(NCH, CH)
    z128 = jnp.zeros((RPS + REXTRA, 128), jnp.float32)
    z64 = jnp.zeros((RPS + REXTRA, 64), jnp.float32)
    z16 = jnp.zeros((RPS + REXTRA, 16), jnp.float32)
    W4p = jnp.pad(W4, ((0, 0), (0, 16 - W4.shape[1])))
    b1r, b2r, b3r, b4r = (b.reshape(1, -1) for b in (b1, b2, b3, b4))

    P1 = _agg128(x, src, dst, z128)
    h1 = pl.pallas_call(
        _tc1_body,
        grid=(_GRID,),
        in_specs=[
            pl.BlockSpec((NC, _BLK, 128), lambda i: (0, i, 0)),
            _full((128, 128)),
            _full((1, 128)),
        ],
        out_specs=pl.BlockSpec((_BLK, 128), lambda i: (i, 0)),
        out_shape=jax.ShapeDtypeStruct((N, 128), jnp.float32),
    )(P1, W1, b1r)

    P2 = _agg128(h1, src, dst, z128)
    t3 = pl.pallas_call(
        _tc2_body,
        grid=(_GRID,),
        in_specs=[
            pl.BlockSpec((NC, _BLK, 128), lambda i: (0, i, 0)),
            _full((128, 128)),
            _full((1, 128)),
            _full((128, 64)),
        ],
        out_specs=pl.BlockSpec((_BLK, 64), lambda i: (i, 0)),
        out_shape=jax.ShapeDtypeStruct((N, 64), jnp.float32),
    )(P2, W2, b2r, W3)

    P3 = _agg64(t3, src, dst, z64)
    t4 = pl.pallas_call(
        _tc3_body,
        grid=(_GRID,),
        in_specs=[
            pl.BlockSpec((NC, _BLK, 64), lambda i: (0, i, 0)),
            _full((1, 64)),
            _full((64, 16)),
        ],
        out_specs=pl.BlockSpec((_BLK, 16), lambda i: (i, 0)),
        out_shape=jax.ShapeDtypeStruct((N, 16), jnp.float32),
    )(P3, b3r, W4p)

    P4 = _agg16(t4, src, dst, z16)
    out = pl.pallas_call(
        _tc4_body,
        grid=(_GRID,),
        in_specs=[
            pl.BlockSpec((NC, _BLK, 16), lambda i: (0, i, 0)),
            _full((1, 2)),
        ],
        out_specs=pl.BlockSpec((_BLK, 2), lambda i: (i, 0)),
        out_shape=jax.ShapeDtypeStruct((N, 2), jnp.float32),
    )(P4, b4r)
    return out
